# baseline (speedup 1.0000x reference)
import functools

import jax
import jax.numpy as jnp
from jax.experimental import pallas as pl
from jax.experimental.pallas import tpu as pltpu
from jax.experimental.pallas import tpu_sc as plsc

_CHUNKS = 8
_IDX_B = 2048
_CMB_B = 256
_GW = 128


def _build_table(svf):
    v = svf[0]
    _, D, H, W = v.shape
    vp = jnp.pad(v, ((0, 0), (0, 1), (0, 1), (0, 1)), mode="edge")
    cols = []
    for c in range(3):
        for k in range(8):
            dz, dy, dx = (k >> 2) & 1, (k >> 1) & 1, k & 1
            cols.append(vp[c, dz:dz + D, dy:dy + H, dx:dx + W].reshape(-1))
    t = jnp.stack(cols, axis=-1)
    return jnp.pad(t, ((0, 0), (0, 104)))


def _pos(xn, D):
    return jnp.clip((xn + 1.0) * (0.5 * (D - 1)), 0.0, float(D - 1))


def _tc_indices(xn3):
    M = xn3.shape[2]
    B = min(_IDX_B, M)

    def body(x_ref, i0_ref, i1_ref):
        xs = x_ref[0], x_ref[1], x_ref[2]

        def flat(D):
            ii = [jnp.floor(_pos(c, D)).astype(jnp.int32) for c in xs]
            return (ii[2] * D + ii[1]) * D + ii[0]

        i0_ref[...] = flat(64)
        i1_ref[...] = flat(128)

    return pl.pallas_call(
        body,
        grid=(M // B,),
        in_specs=[pl.BlockSpec((3, 8, B), lambda w: (0, 0, w))],
        out_specs=[pl.BlockSpec((8, B), lambda w: (0, w)),
                   pl.BlockSpec((8, B), lambda w: (0, w))],
        out_shape=[jax.ShapeDtypeStruct((8, M), jnp.int32),
                   jax.ShapeDtypeStruct((8, M), jnp.int32)],
    )(xn3)


def _sc_gather(t0, i0, t1, i1):
    M = i0.shape[1]
    nw = M // _GW
    mesh = plsc.VectorSubcoreMesh(core_axis_name="core",
                                  subcore_axis_name="subcore")

    @functools.partial(
        pl.kernel,
        out_type=[jax.ShapeDtypeStruct((8, M, 128), jnp.float32),
                  jax.ShapeDtypeStruct((8, M, 128), jnp.float32)],
        mesh=mesh)
    def k(t0_hbm, i0_hbm, t1_hbm, i1_hbm, o0_hbm, o1_hbm):
        def body(i0_v, i1_v, o0_v, o1_v):
            pltpu.sync_copy(t0_hbm.at[i0_v.at[0]], o0_v.at[0])
            pltpu.sync_copy(t1_hbm.at[i1_v.at[0]], o1_v.at[0])

        pltpu.emit_pipeline(
            body,
            grid=(8 * nw,),
            in_specs=[pl.BlockSpec((1, _GW), lambda i: (i // nw, i % nw)),
                      pl.BlockSpec((1, _GW), lambda i: (i // nw, i % nw))],
            out_specs=[pl.BlockSpec((1, _GW, 128),
                                    lambda i: (i // nw, i % nw, 0)),
                       pl.BlockSpec((1, _GW, 128),
                                    lambda i: (i // nw, i % nw, 0))],
            core_axis_name=("core", "subcore"),
            dimension_semantics=(pltpu.PARALLEL,),
        )(i0_hbm, i1_hbm, o0_hbm, o1_hbm)

    return k(t0, i0, t1, i1)


def _tc_combine(xn3, g0, g1):
    M = xn3.shape[2]
    B = min(_CMB_B, M)

    def body(x_ref, g0_ref, g1_ref, ox_ref, oy_ref, oz_ref):
        xs = x_ref[0], x_ref[1], x_ref[2]
        acc = [jnp.zeros((8, B), jnp.float32) for _ in range(3)]
        for g_ref, D in ((g0_ref, 64), (g1_ref, 128)):
            pos = [_pos(c, D) for c in xs]
            fr = [p - jnp.floor(p) for p in pos]
            wx, wy, wz = fr
            t = jnp.transpose(g_ref[:, :, :32], (0, 2, 1))
            az = (1.0 - wz, wz)
            ay = (1.0 - wy, wy)
            ax = (1.0 - wx, wx)
            for dz in range(2):
                for dy in range(2):
                    zy = az[dz] * ay[dy]
                    for dx in range(2):
                        w = zy * ax[dx]
                        k = dz * 4 + dy * 2 + dx
                        for c in range(3):
                            acc[c] = acc[c] + w * t[:, 8 * c + k, :]
        ox_ref[...] = acc[0]
        oy_ref[...] = acc[1]
        oz_ref[...] = acc[2]

    return pl.pallas_call(
        body,
        grid=(M // B,),
        in_specs=[pl.BlockSpec((3, 8, B), lambda w: (0, 0, w)),
                  pl.BlockSpec((8, B, 128), lambda w: (0, w, 0)),
                  pl.BlockSpec((8, B, 128), lambda w: (0, w, 0))],
        out_specs=[pl.BlockSpec((8, B), lambda w: (0, w)),
                   pl.BlockSpec((8, B), lambda w: (0, w)),
                   pl.BlockSpec((8, B), lambda w: (0, w))],
        out_shape=[jax.ShapeDtypeStruct((8, M), jnp.float32),
                   jax.ShapeDtypeStruct((8, M), jnp.float32),
                   jax.ShapeDtypeStruct((8, M), jnp.float32)],
    )(xn3, g0, g1)


def kernel(x_world, center, half, svf_L0, svf_L1):
    G = x_world.shape[0]
    C = _CHUNKS
    M = G // (8 * C)
    t0 = _build_table(svf_L0)
    t1 = _build_table(svf_L1)
    invh = 1.0 / (half + 1e-8)
    xn = jnp.clip((x_world - center[None, :]) * invh[None, :], -1.5, 1.5)
    xn3 = jnp.transpose(xn, (1, 0)).reshape(3, C, 8, M)
    outs = []
    for ci in range(C):
        xc = xn3[:, ci]
        i0, i1 = _tc_indices(xc)
        g0, g1 = _sc_gather(t0, i0, t1, i1)
        outs.append(_tc_combine(xc, g0, g1))
    ox = jnp.concatenate([o[0].reshape(-1) for o in outs])
    oy = jnp.concatenate([o[1].reshape(-1) for o in outs])
    oz = jnp.concatenate([o[2].reshape(-1) for o in outs])
    return jnp.stack([ox, oy, oz], axis=1)


# speedup vs baseline: 1.4456x; 1.4456x over previous
import functools

import jax
import jax.numpy as jnp
from jax.experimental import pallas as pl
from jax.experimental.pallas import tpu as pltpu
from jax.experimental.pallas import tpu_sc as plsc

_CHUNKS = 8
_IDX_B = 2048
_CMB_B = 256
_GW = 128


def _build_table(svf):
    v = svf[0]
    _, D, H, W = v.shape
    vp = jnp.pad(v, ((0, 0), (0, 1), (0, 1), (0, 1)), mode="edge")
    cols = []
    for c in range(3):
        for k in range(8):
            dz, dy, dx = (k >> 2) & 1, (k >> 1) & 1, k & 1
            cols.append(vp[c, dz:dz + D, dy:dy + H, dx:dx + W].reshape(-1))
    t = jnp.stack(cols, axis=-1)
    return jnp.pad(t, ((0, 0), (0, 104)))


def _pos(xn, D):
    return jnp.clip((xn + 1.0) * (0.5 * (D - 1)), 0.0, float(D - 1))


def _tc_indices(xn3):
    M = xn3.shape[2]
    B = min(_IDX_B, M)

    def body(x_ref, i0_ref, i1_ref):
        xs = x_ref[0], x_ref[1], x_ref[2]

        def flat(D):
            ii = [jnp.floor(_pos(c, D)).astype(jnp.int32) for c in xs]
            return (ii[2] * D + ii[1]) * D + ii[0]

        i0_ref[...] = flat(64)
        i1_ref[...] = flat(128)

    return pl.pallas_call(
        body,
        grid=(M // B,),
        in_specs=[pl.BlockSpec((3, 8, B), lambda w: (0, 0, w))],
        out_specs=[pl.BlockSpec((8, B), lambda w: (0, w)),
                   pl.BlockSpec((8, B), lambda w: (0, w))],
        out_shape=[jax.ShapeDtypeStruct((8, M), jnp.int32),
                   jax.ShapeDtypeStruct((8, M), jnp.int32)],
    )(xn3)


def _sc_gather(t0, i0, t1, i1):
    M = i0.shape[1]
    nw = M // _GW
    mesh = plsc.VectorSubcoreMesh(core_axis_name="core",
                                  subcore_axis_name="subcore")

    @functools.partial(
        pl.kernel,
        out_type=[jax.ShapeDtypeStruct((8, M, 128), jnp.float32),
                  jax.ShapeDtypeStruct((8, M, 128), jnp.float32)],
        scratch_types=[pltpu.SemaphoreType.DMA, pltpu.SemaphoreType.DMA],
        mesh=mesh)
    def k(t0_hbm, i0_hbm, t1_hbm, i1_hbm, o0_hbm, o1_hbm, sem0, sem1):
        def body(i0_v, i1_v, o0_v, o1_v):
            c0 = pltpu.async_copy(t0_hbm.at[i0_v.at[0]], o0_v.at[0], sem0)
            c1 = pltpu.async_copy(t1_hbm.at[i1_v.at[0]], o1_v.at[0], sem1)
            c0.wait()
            c1.wait()

        pltpu.emit_pipeline(
            body,
            grid=(8 * nw,),
            in_specs=[pl.BlockSpec((1, _GW), lambda i: (i // nw, i % nw)),
                      pl.BlockSpec((1, _GW), lambda i: (i // nw, i % nw))],
            out_specs=[pl.BlockSpec((1, _GW, 128),
                                    lambda i: (i // nw, i % nw, 0)),
                       pl.BlockSpec((1, _GW, 128),
                                    lambda i: (i // nw, i % nw, 0))],
            core_axis_name=("core", "subcore"),
            dimension_semantics=(pltpu.PARALLEL,),
        )(i0_hbm, i1_hbm, o0_hbm, o1_hbm)

    return k(t0, i0, t1, i1)


def _tc_combine(xn3, g0, g1):
    M = xn3.shape[2]
    B = min(_CMB_B, M)

    def body(x_ref, g0_ref, g1_ref, ox_ref, oy_ref, oz_ref):
        xs = x_ref[0], x_ref[1], x_ref[2]
        acc = [jnp.zeros((8, B), jnp.float32) for _ in range(3)]
        for g_ref, D in ((g0_ref, 64), (g1_ref, 128)):
            pos = [_pos(c, D) for c in xs]
            fr = [p - jnp.floor(p) for p in pos]
            wx, wy, wz = fr
            t = jnp.transpose(g_ref[:, :, :32], (0, 2, 1))
            az = (1.0 - wz, wz)
            ay = (1.0 - wy, wy)
            ax = (1.0 - wx, wx)
            for dz in range(2):
                for dy in range(2):
                    zy = az[dz] * ay[dy]
                    for dx in range(2):
                        w = zy * ax[dx]
                        k = dz * 4 + dy * 2 + dx
                        for c in range(3):
                            acc[c] = acc[c] + w * t[:, 8 * c + k, :]
        ox_ref[...] = acc[0]
        oy_ref[...] = acc[1]
        oz_ref[...] = acc[2]

    return pl.pallas_call(
        body,
        grid=(M // B,),
        in_specs=[pl.BlockSpec((3, 8, B), lambda w: (0, 0, w)),
                  pl.BlockSpec((8, B, 128), lambda w: (0, w, 0)),
                  pl.BlockSpec((8, B, 128), lambda w: (0, w, 0))],
        out_specs=[pl.BlockSpec((8, B), lambda w: (0, w)),
                   pl.BlockSpec((8, B), lambda w: (0, w)),
                   pl.BlockSpec((8, B), lambda w: (0, w))],
        out_shape=[jax.ShapeDtypeStruct((8, M), jnp.float32),
                   jax.ShapeDtypeStruct((8, M), jnp.float32),
                   jax.ShapeDtypeStruct((8, M), jnp.float32)],
    )(xn3, g0, g1)


def kernel(x_world, center, half, svf_L0, svf_L1):
    G = x_world.shape[0]
    C = _CHUNKS
    M = G // (8 * C)
    t0 = _build_table(svf_L0)
    t1 = _build_table(svf_L1)
    invh = 1.0 / (half + 1e-8)
    xn = jnp.clip((x_world - center[None, :]) * invh[None, :], -1.5, 1.5)
    xn3 = jnp.transpose(xn, (1, 0)).reshape(3, C, 8, M)
    outs = []
    for ci in range(C):
        xc = xn3[:, ci]
        i0, i1 = _tc_indices(xc)
        g0, g1 = _sc_gather(t0, i0, t1, i1)
        outs.append(_tc_combine(xc, g0, g1))
    ox = jnp.concatenate([o[0].reshape(-1) for o in outs])
    oy = jnp.concatenate([o[1].reshape(-1) for o in outs])
    oz = jnp.concatenate([o[2].reshape(-1) for o in outs])
    return jnp.stack([ox, oy, oz], axis=1)
